# baseline (speedup 1.0000x reference)
import functools

import jax
import jax.numpy as jnp
from jax import lax
from jax.experimental import pallas as pl
from jax.experimental.pallas import tpu as pltpu
from jax.experimental.pallas import tpu_sc as plsc

N = 10000
NP = 10240
E = 320000
D = 128

NC = 2
NS = 16
NW = NC * NS
EPW = E // NW
K = 128
NCHUNK = 79
EPP = NCHUNK * K
RPT = NP // NS

_MESH = plsc.VectorSubcoreMesh(core_axis_name="c", subcore_axis_name="s")


def _sc_prop_body(msg_hbm, edge_hbm, out_hbm, dst_v, s0, s1, rows0, rows1,
                  acc, si0, si1, sg0, sg1, ss0, ss1):
    c = lax.axis_index("c")
    s = lax.axis_index("s")
    wid = s * NC + c

    pltpu.sync_copy(edge_hbm.at[1, wid], dst_v)
    pltpu.sync_copy(edge_hbm.at[0, wid, 0], s0)
    pltpu.sync_copy(edge_hbm.at[0, wid, 1], s1)

    def zrow(i, _):
        for j in range(D // 16):
            rows0[i, pl.ds(j * 16, 16)] = jnp.zeros((16,), jnp.float32)
        return 0
    lax.fori_loop(0, K, zrow, 0)

    rowbase = s * RPT
    for b in range(RPT // K):
        pltpu.sync_copy(rows0, acc.at[pl.ds(rowbase + b * K, K), :])

    pltpu.async_copy(msg_hbm.at[s0], rows0, sg0)
    pltpu.async_copy(msg_hbm.at[s1], rows1, sg1)
    plsc.subcore_barrier()

    def step(j, sbuf, rows, si, sg, ss, nxt):
        pltpu.make_async_copy(msg_hbm.at[sbuf], rows, sg).wait()
        if nxt:
            pltpu.async_copy(edge_hbm.at[0, wid, j + 2], sbuf, si)
        pltpu.sync_copy(rows, acc.at[dst_v.at[j]], add=True)
        if nxt:
            pltpu.make_async_copy(edge_hbm.at[0, wid, 0], sbuf, si).wait()
            pltpu.async_copy(msg_hbm.at[sbuf], rows, sg)

    def pair(i, _):
        j0 = 2 * i
        step(j0, s0, rows0, si0, sg0, ss0, True)
        step(j0 + 1, s1, rows1, si1, sg1, ss1, False)

        @pl.when(i < (NCHUNK - 1) // 2 - 1)
        def _():
            pltpu.async_copy(edge_hbm.at[0, wid, j0 + 3], s1, si1)
            pltpu.make_async_copy(edge_hbm.at[0, wid, 0], s1, si1).wait()
            pltpu.async_copy(msg_hbm.at[s1], rows1, sg1)
        return 0
    lax.fori_loop(0, (NCHUNK - 1) // 2, pair, 0)

    pltpu.make_async_copy(msg_hbm.at[s0], rows0, sg0).wait()
    pltpu.sync_copy(rows0, acc.at[dst_v.at[NCHUNK - 1]], add=True)

    plsc.subcore_barrier()

    pltpu.sync_copy(acc.at[pl.ds(rowbase, RPT), :],
                    out_hbm.at[c, pl.ds(rowbase, RPT), :])


_sc_prop = pl.kernel(
    _sc_prop_body,
    out_type=(jax.ShapeDtypeStruct((NC, NP, D), jnp.float32),),
    mesh=_MESH,
    scratch_types=[
        pltpu.VMEM((NCHUNK, K), jnp.int32),
        pltpu.VMEM((K,), jnp.int32),
        pltpu.VMEM((K,), jnp.int32),
        pltpu.VMEM((K, D), jnp.float32),
        pltpu.VMEM((K, D), jnp.float32),
        pltpu.VMEM_SHARED((NP, D), jnp.float32),
        pltpu.SemaphoreType.DMA,
        pltpu.SemaphoreType.DMA,
        pltpu.SemaphoreType.DMA,
        pltpu.SemaphoreType.DMA,
        pltpu.SemaphoreType.DMA,
        pltpu.SemaphoreType.DMA,
    ],
)


def _sc_deg_body(edge_hbm, deg_hbm, dst_v, ones_v, acc16):
    c = lax.axis_index("c")
    s = lax.axis_index("s")
    wid = s * NC + c

    def zrow(i, _):
        for j in range(D // 16):
            ones_v[i, pl.ds(j * 16, 16)] = jnp.zeros((16,), jnp.float32)
        return 0
    lax.fori_loop(0, K, zrow, 0)

    rowbase = s * RPT
    for b in range(RPT // K):
        pltpu.sync_copy(ones_v, acc16.at[pl.ds(rowbase + b * K, K), :])

    def orow(i, _):
        for j in range(D // 16):
            ones_v[i, pl.ds(j * 16, 16)] = jnp.ones((16,), jnp.float32)
        return 0
    lax.fori_loop(0, K, orow, 0)
    plsc.subcore_barrier()

    pltpu.sync_copy(edge_hbm.at[1, wid], dst_v)

    def chunk(j, _):
        pltpu.sync_copy(ones_v, acc16.at[dst_v.at[j]], add=True)
        return 0
    lax.fori_loop(0, NCHUNK, chunk, 0)

    plsc.subcore_barrier()
    pltpu.sync_copy(acc16.at[pl.ds(rowbase, RPT), :],
                    deg_hbm.at[c, pl.ds(rowbase, RPT), :])


_sc_deg = pl.kernel(
    _sc_deg_body,
    out_type=(jax.ShapeDtypeStruct((NC, NP, D), jnp.float32),),
    mesh=_MESH,
    scratch_types=[
        pltpu.VMEM((NCHUNK, K), jnp.int32),
        pltpu.VMEM((K, D), jnp.float32),
        pltpu.VMEM_SHARED((NP, D), jnp.float32),
    ],
)


BN = 2048
_G = NP // BN


def _tc_in_body(x_ref, w1_ref, b1_ref, w2_ref, b2_ref, feat_ref):
    h = jnp.dot(x_ref[...], w1_ref[...],
                preferred_element_type=jnp.float32) + b1_ref[...]
    h = h * jax.nn.sigmoid(h)
    feat_ref[...] = jnp.dot(h, w2_ref[...],
                            preferred_element_type=jnp.float32) + b2_ref[...]


def _tc_in(x, w1, b1, w2, b2):
    return pl.pallas_call(
        _tc_in_body,
        grid=(_G,),
        in_specs=[
            pl.BlockSpec((BN, D), lambda i: (i, 0)),
            pl.BlockSpec((D, D), lambda i: (0, 0)),
            pl.BlockSpec((1, D), lambda i: (0, 0)),
            pl.BlockSpec((D, D), lambda i: (0, 0)),
            pl.BlockSpec((1, D), lambda i: (0, 0)),
        ],
        out_specs=pl.BlockSpec((BN, D), lambda i: (i, 0)),
        out_shape=jax.ShapeDtypeStruct((NP, D), jnp.float32),
    )(x, w1, b1, w2, b2)


def _tc_comb_body(sum_deg, first, p_ref, deg_ref, w_ref, *rest):
    rest = list(rest)
    accin_ref = None if first else rest.pop(0)
    msg_ref, acc_ref = rest[0], rest[1]
    degout_ref = rest[2] if sum_deg else None

    if sum_deg:
        deg = deg_ref[0, :, :16] + deg_ref[1, :, :16]
        degout_ref[...] = deg
    else:
        deg = deg_ref[...]
    inv = 1.0 / jnp.maximum(deg[:, :1], 1.0)
    msg = (p_ref[0] + p_ref[1]) * inv
    msg_ref[...] = msg
    t = jnp.dot(msg, w_ref[...], preferred_element_type=jnp.float32)
    t = t * jax.nn.sigmoid(t)
    acc_ref[...] = t if first else accin_ref[...] + t


def _tc_combine(p, deg_or_degp, w, acc, sum_deg):
    first = acc is None
    in_specs = [
        pl.BlockSpec((NC, BN, D), lambda i: (0, i, 0)),
        (pl.BlockSpec((NC, BN, D), lambda i: (0, i, 0)) if sum_deg
         else pl.BlockSpec((BN, 16), lambda i: (i, 0))),
        pl.BlockSpec((D, D), lambda i: (0, 0)),
    ]
    args = [p, deg_or_degp, w]
    if not first:
        in_specs.append(pl.BlockSpec((BN, D), lambda i: (i, 0)))
        args.append(acc)
    out_specs = [
        pl.BlockSpec((BN, D), lambda i: (i, 0)),
        pl.BlockSpec((BN, D), lambda i: (i, 0)),
    ]
    out_shape = [
        jax.ShapeDtypeStruct((NP, D), jnp.float32),
        jax.ShapeDtypeStruct((NP, D), jnp.float32),
    ]
    if sum_deg:
        out_specs.append(pl.BlockSpec((BN, 16), lambda i: (i, 0)))
        out_shape.append(jax.ShapeDtypeStruct((NP, 16), jnp.float32))
    return pl.pallas_call(
        functools.partial(_tc_comb_body, sum_deg, first),
        grid=(_G,),
        in_specs=in_specs,
        out_specs=out_specs,
        out_shape=out_shape,
    )(*args)


def _tc_out_body(last, feat_ref, acc_ref, w_ref, b_ref, out_ref):
    h = jnp.dot(feat_ref[...] + acc_ref[...], w_ref[...],
                preferred_element_type=jnp.float32) + b_ref[...]
    out_ref[...] = h if last else h * jax.nn.sigmoid(h)


def _tc_out(feat, acc, w, b, last):
    return pl.pallas_call(
        functools.partial(_tc_out_body, last),
        grid=(_G,),
        in_specs=[
            pl.BlockSpec((BN, D), lambda i: (i, 0)),
            pl.BlockSpec((BN, D), lambda i: (i, 0)),
            pl.BlockSpec((D, D), lambda i: (0, 0)),
            pl.BlockSpec((1, D), lambda i: (0, 0)),
        ],
        out_specs=pl.BlockSpec((BN, D), lambda i: (i, 0)),
        out_shape=jax.ShapeDtypeStruct((NP, D), jnp.float32),
    )(feat, acc, w, b)


def kernel(x, edge_index, W_in1, b_in1, W_in2, b_in2, W_up,
           W_out0, b_out0, W_out1, b_out1):
    er = edge_index.reshape(2, NW, EPW)
    npad = EPP - EPW
    pad_src = jnp.full((NW, npad), NP - 1, dtype=edge_index.dtype)
    lane = jnp.arange(npad, dtype=edge_index.dtype)[None, :]
    tile = jnp.arange(NW, dtype=edge_index.dtype)[:, None]
    pad_dst = N + (lane + tile * 7) % (NP - N)
    pad = jnp.stack([pad_src, pad_dst]).astype(edge_index.dtype)
    edge4 = jnp.concatenate([er, pad], axis=2).reshape(2, NW, NCHUNK, K)
    xp = jnp.pad(x, ((0, NP - N), (0, 0)))
    b1 = b_in1.reshape(1, D)
    b2 = b_in2.reshape(1, D)
    bo0 = b_out0.reshape(1, D)
    bo1 = b_out1.reshape(1, D)

    feat = _tc_in(xp, W_in1, b1, W_in2, b2)
    (degp,) = _sc_deg(edge4)

    deg = None
    for layer in range(2):
        w_out, b_out = (W_out0, bo0) if layer == 0 else (W_out1, bo1)
        msg, acc = feat, None
        for level in range(4):
            (p,) = _sc_prop(msg, edge4)
            if deg is None:
                msg, acc, deg = _tc_combine(p, degp, W_up[layer, level],
                                            acc, sum_deg=True)
            else:
                msg, acc = _tc_combine(p, deg, W_up[layer, level],
                                       acc, sum_deg=False)
        feat = _tc_out(feat, acc, w_out, b_out, last=(layer == 1))
    return feat[:N]


# speedup vs baseline: 1.6587x; 1.6587x over previous
import functools

import jax
import jax.numpy as jnp
from jax import lax
from jax.experimental import pallas as pl
from jax.experimental.pallas import tpu as pltpu
from jax.experimental.pallas import tpu_sc as plsc

N = 10000
NP = 10240
E = 320000
D = 128

NC = 2
NS = 16
NW = NC * NS
EPW = E // NW
K = 80
NCHUNK = EPW // K
RPT = NP // NS

_MESH = plsc.VectorSubcoreMesh(core_axis_name="c", subcore_axis_name="s")


def _sc_prop_body(msg_hbm, edge_hbm, out_hbm, dst_v, s0, s1, rows0, rows1,
                  acc, si0, si1, sg0, sg1, ss0, ss1):
    c = lax.axis_index("c")
    s = lax.axis_index("s")
    wid = s * NC + c

    pltpu.sync_copy(edge_hbm.at[1, wid], dst_v)
    pltpu.sync_copy(edge_hbm.at[0, wid, 0], s0)
    pltpu.sync_copy(edge_hbm.at[0, wid, 1], s1)

    def zrow(i, _):
        for j in range(D // 16):
            rows0[i, pl.ds(j * 16, 16)] = jnp.zeros((16,), jnp.float32)
        return 0
    lax.fori_loop(0, K, zrow, 0)

    rowbase = s * RPT
    for b in range(RPT // K):
        pltpu.sync_copy(rows0, acc.at[pl.ds(rowbase + b * K, K), :])

    pltpu.async_copy(msg_hbm.at[s0], rows0, sg0)
    pltpu.async_copy(msg_hbm.at[s1], rows1, sg1)
    plsc.subcore_barrier()

    def step(j, sbuf, rows, si, sg, ss, nxt):
        pltpu.make_async_copy(msg_hbm.at[sbuf], rows, sg).wait()
        if nxt:
            pltpu.async_copy(edge_hbm.at[0, wid, j + 2], sbuf, si)
        pltpu.sync_copy(rows, acc.at[dst_v.at[j]], add=True)
        if nxt:
            pltpu.make_async_copy(edge_hbm.at[0, wid, 0], sbuf, si).wait()
            pltpu.async_copy(msg_hbm.at[sbuf], rows, sg)

    def pair(i, _):
        j0 = 2 * i
        step(j0, s0, rows0, si0, sg0, ss0, True)
        step(j0 + 1, s1, rows1, si1, sg1, ss1, False)

        @pl.when(i < (NCHUNK - 1) // 2 - 1)
        def _():
            pltpu.async_copy(edge_hbm.at[0, wid, j0 + 3], s1, si1)
            pltpu.make_async_copy(edge_hbm.at[0, wid, 0], s1, si1).wait()
            pltpu.async_copy(msg_hbm.at[s1], rows1, sg1)
        return 0
    lax.fori_loop(0, (NCHUNK - 1) // 2, pair, 0)

    pltpu.make_async_copy(msg_hbm.at[s0], rows0, sg0).wait()
    pltpu.sync_copy(rows0, acc.at[dst_v.at[NCHUNK - 1]], add=True)

    plsc.subcore_barrier()

    pltpu.sync_copy(acc.at[pl.ds(rowbase, RPT), :],
                    out_hbm.at[c, pl.ds(rowbase, RPT), :])


_sc_prop = pl.kernel(
    _sc_prop_body,
    out_type=(jax.ShapeDtypeStruct((NC, NP, D), jnp.float32),),
    mesh=_MESH,
    scratch_types=[
        pltpu.VMEM((NCHUNK, K), jnp.int32),
        pltpu.VMEM((K,), jnp.int32),
        pltpu.VMEM((K,), jnp.int32),
        pltpu.VMEM((K, D), jnp.float32),
        pltpu.VMEM((K, D), jnp.float32),
        pltpu.VMEM_SHARED((NP, D), jnp.float32),
        pltpu.SemaphoreType.DMA,
        pltpu.SemaphoreType.DMA,
        pltpu.SemaphoreType.DMA,
        pltpu.SemaphoreType.DMA,
        pltpu.SemaphoreType.DMA,
        pltpu.SemaphoreType.DMA,
    ],
)


def _sc_deg_body(edge_hbm, deg_hbm, dst_v, ones_v, acc16):
    c = lax.axis_index("c")
    s = lax.axis_index("s")
    wid = s * NC + c

    def zrow(i, _):
        for j in range(D // 16):
            ones_v[i, pl.ds(j * 16, 16)] = jnp.zeros((16,), jnp.float32)
        return 0
    lax.fori_loop(0, K, zrow, 0)

    rowbase = s * RPT
    for b in range(RPT // K):
        pltpu.sync_copy(ones_v, acc16.at[pl.ds(rowbase + b * K, K), :])

    def orow(i, _):
        for j in range(D // 16):
            ones_v[i, pl.ds(j * 16, 16)] = jnp.ones((16,), jnp.float32)
        return 0
    lax.fori_loop(0, K, orow, 0)
    plsc.subcore_barrier()

    pltpu.sync_copy(edge_hbm.at[1, wid], dst_v)

    def chunk(j, _):
        pltpu.sync_copy(ones_v, acc16.at[dst_v.at[j]], add=True)
        return 0
    lax.fori_loop(0, NCHUNK, chunk, 0)

    plsc.subcore_barrier()
    pltpu.sync_copy(acc16.at[pl.ds(rowbase, RPT), :],
                    deg_hbm.at[c, pl.ds(rowbase, RPT), :])


_sc_deg = pl.kernel(
    _sc_deg_body,
    out_type=(jax.ShapeDtypeStruct((NC, NP, D), jnp.float32),),
    mesh=_MESH,
    scratch_types=[
        pltpu.VMEM((NCHUNK, K), jnp.int32),
        pltpu.VMEM((K, D), jnp.float32),
        pltpu.VMEM_SHARED((NP, D), jnp.float32),
    ],
)


BN = 2048
_G = NP // BN


def _tc_in_body(x_ref, w1_ref, b1_ref, w2_ref, b2_ref, feat_ref):
    h = jnp.dot(x_ref[...], w1_ref[...],
                preferred_element_type=jnp.float32) + b1_ref[...]
    h = h * jax.nn.sigmoid(h)
    feat_ref[...] = jnp.dot(h, w2_ref[...],
                            preferred_element_type=jnp.float32) + b2_ref[...]


def _tc_in(x, w1, b1, w2, b2):
    return pl.pallas_call(
        _tc_in_body,
        grid=(_G,),
        in_specs=[
            pl.BlockSpec((BN, D), lambda i: (i, 0)),
            pl.BlockSpec((D, D), lambda i: (0, 0)),
            pl.BlockSpec((1, D), lambda i: (0, 0)),
            pl.BlockSpec((D, D), lambda i: (0, 0)),
            pl.BlockSpec((1, D), lambda i: (0, 0)),
        ],
        out_specs=pl.BlockSpec((BN, D), lambda i: (i, 0)),
        out_shape=jax.ShapeDtypeStruct((NP, D), jnp.float32),
    )(x, w1, b1, w2, b2)


def _tc_comb_body(sum_deg, first, p_ref, deg_ref, w_ref, *rest):
    rest = list(rest)
    accin_ref = None if first else rest.pop(0)
    msg_ref, acc_ref = rest[0], rest[1]
    degout_ref = rest[2] if sum_deg else None

    if sum_deg:
        deg = deg_ref[0, :, :16] + deg_ref[1, :, :16]
        degout_ref[...] = deg
    else:
        deg = deg_ref[...]
    inv = 1.0 / jnp.maximum(deg[:, :1], 1.0)
    msg = (p_ref[0] + p_ref[1]) * inv
    msg_ref[...] = msg
    t = jnp.dot(msg, w_ref[...], preferred_element_type=jnp.float32)
    t = t * jax.nn.sigmoid(t)
    acc_ref[...] = t if first else accin_ref[...] + t


def _tc_combine(p, deg_or_degp, w, acc, sum_deg):
    first = acc is None
    in_specs = [
        pl.BlockSpec((NC, BN, D), lambda i: (0, i, 0)),
        (pl.BlockSpec((NC, BN, D), lambda i: (0, i, 0)) if sum_deg
         else pl.BlockSpec((BN, 16), lambda i: (i, 0))),
        pl.BlockSpec((D, D), lambda i: (0, 0)),
    ]
    args = [p, deg_or_degp, w]
    if not first:
        in_specs.append(pl.BlockSpec((BN, D), lambda i: (i, 0)))
        args.append(acc)
    out_specs = [
        pl.BlockSpec((BN, D), lambda i: (i, 0)),
        pl.BlockSpec((BN, D), lambda i: (i, 0)),
    ]
    out_shape = [
        jax.ShapeDtypeStruct((NP, D), jnp.float32),
        jax.ShapeDtypeStruct((NP, D), jnp.float32),
    ]
    if sum_deg:
        out_specs.append(pl.BlockSpec((BN, 16), lambda i: (i, 0)))
        out_shape.append(jax.ShapeDtypeStruct((NP, 16), jnp.float32))
    return pl.pallas_call(
        functools.partial(_tc_comb_body, sum_deg, first),
        grid=(_G,),
        in_specs=in_specs,
        out_specs=out_specs,
        out_shape=out_shape,
    )(*args)


def _tc_out_body(last, feat_ref, acc_ref, w_ref, b_ref, out_ref):
    h = jnp.dot(feat_ref[...] + acc_ref[...], w_ref[...],
                preferred_element_type=jnp.float32) + b_ref[...]
    out_ref[...] = h if last else h * jax.nn.sigmoid(h)


def _tc_out(feat, acc, w, b, last):
    return pl.pallas_call(
        functools.partial(_tc_out_body, last),
        grid=(_G,),
        in_specs=[
            pl.BlockSpec((BN, D), lambda i: (i, 0)),
            pl.BlockSpec((BN, D), lambda i: (i, 0)),
            pl.BlockSpec((D, D), lambda i: (0, 0)),
            pl.BlockSpec((1, D), lambda i: (0, 0)),
        ],
        out_specs=pl.BlockSpec((BN, D), lambda i: (i, 0)),
        out_shape=jax.ShapeDtypeStruct((NP, D), jnp.float32),
    )(feat, acc, w, b)


def kernel(x, edge_index, W_in1, b_in1, W_in2, b_in2, W_up,
           W_out0, b_out0, W_out1, b_out1):
    edge4 = edge_index.reshape(2, NW, NCHUNK, K)
    xp = jnp.pad(x, ((0, NP - N), (0, 0)))
    b1 = b_in1.reshape(1, D)
    b2 = b_in2.reshape(1, D)
    bo0 = b_out0.reshape(1, D)
    bo1 = b_out1.reshape(1, D)

    feat = _tc_in(xp, W_in1, b1, W_in2, b2)
    (degp,) = _sc_deg(edge4)

    deg = None
    for layer in range(2):
        w_out, b_out = (W_out0, bo0) if layer == 0 else (W_out1, bo1)
        msg, acc = feat, None
        for level in range(4):
            (p,) = _sc_prop(msg, edge4)
            if deg is None:
                msg, acc, deg = _tc_combine(p, degp, W_up[layer, level],
                                            acc, sum_deg=True)
            else:
                msg, acc = _tc_combine(p, deg, W_up[layer, level],
                                       acc, sum_deg=False)
        feat = _tc_out(feat, acc, w_out, b_out, last=(layer == 1))
    return feat[:N]


# speedup vs baseline: 1.6751x; 1.0099x over previous
import functools

import jax
import jax.numpy as jnp
from jax import lax
from jax.experimental import pallas as pl
from jax.experimental.pallas import tpu as pltpu
from jax.experimental.pallas import tpu_sc as plsc

N = 10000
NP = 10240
E = 320000
D = 128

NC = 2
NS = 16
NW = NC * NS
EPW = E // NW
K = 80
NCHUNK = EPW // K
RPT = NP // NS

_MESH = plsc.VectorSubcoreMesh(core_axis_name="c", subcore_axis_name="s")


def _sc_prop_body(msg_hbm, edge_hbm, out_hbm, dst_v, s0, s1, rows0, rows1,
                  acc, si0, si1, sg0, sg1, ss0, ss1):
    c = lax.axis_index("c")
    s = lax.axis_index("s")
    wid = s * NC + c

    pltpu.sync_copy(edge_hbm.at[1, wid], dst_v)
    pltpu.sync_copy(edge_hbm.at[0, wid, 0], s0)
    pltpu.sync_copy(edge_hbm.at[0, wid, 1], s1)

    def zrow(i, _):
        for j in range(D // 16):
            rows0[i, pl.ds(j * 16, 16)] = jnp.zeros((16,), jnp.float32)
        return 0
    lax.fori_loop(0, K, zrow, 0)

    rowbase = s * RPT
    for b in range(RPT // K):
        pltpu.sync_copy(rows0, acc.at[pl.ds(rowbase + b * K, K), :])

    pltpu.async_copy(msg_hbm.at[s0], rows0, sg0)
    pltpu.async_copy(msg_hbm.at[s1], rows1, sg1)
    plsc.subcore_barrier()

    def step(j, sbuf, rows, si, sg, ss, nxt):
        pltpu.make_async_copy(msg_hbm.at[sbuf], rows, sg).wait()
        if nxt:
            pltpu.async_copy(edge_hbm.at[0, wid, j + 2], sbuf, si)
        pltpu.sync_copy(rows, acc.at[dst_v.at[j]], add=True)
        if nxt:
            pltpu.make_async_copy(edge_hbm.at[0, wid, 0], sbuf, si).wait()
            pltpu.async_copy(msg_hbm.at[sbuf], rows, sg)

    def pair(i, _):
        j0 = 2 * i
        step(j0, s0, rows0, si0, sg0, ss0, True)
        step(j0 + 1, s1, rows1, si1, sg1, ss1, False)

        @pl.when(i < (NCHUNK - 1) // 2 - 1)
        def _():
            pltpu.async_copy(edge_hbm.at[0, wid, j0 + 3], s1, si1)
            pltpu.make_async_copy(edge_hbm.at[0, wid, 0], s1, si1).wait()
            pltpu.async_copy(msg_hbm.at[s1], rows1, sg1)
        return 0
    lax.fori_loop(0, (NCHUNK - 1) // 2, pair, 0)

    pltpu.make_async_copy(msg_hbm.at[s0], rows0, sg0).wait()
    pltpu.sync_copy(rows0, acc.at[dst_v.at[NCHUNK - 1]], add=True)

    plsc.subcore_barrier()

    pltpu.sync_copy(acc.at[pl.ds(rowbase, RPT), :],
                    out_hbm.at[c, pl.ds(rowbase, RPT), :])


_sc_prop = pl.kernel(
    _sc_prop_body,
    out_type=(jax.ShapeDtypeStruct((NC, NP, D), jnp.float32),),
    mesh=_MESH,
    scratch_types=[
        pltpu.VMEM((NCHUNK, K), jnp.int32),
        pltpu.VMEM((K,), jnp.int32),
        pltpu.VMEM((K,), jnp.int32),
        pltpu.VMEM((K, D), jnp.float32),
        pltpu.VMEM((K, D), jnp.float32),
        pltpu.VMEM_SHARED((NP, D), jnp.float32),
        pltpu.SemaphoreType.DMA,
        pltpu.SemaphoreType.DMA,
        pltpu.SemaphoreType.DMA,
        pltpu.SemaphoreType.DMA,
        pltpu.SemaphoreType.DMA,
        pltpu.SemaphoreType.DMA,
    ],
)


def _sc_deg_body(edge_hbm, deg_hbm, dst_v, ones_v, acc16):
    c = lax.axis_index("c")
    s = lax.axis_index("s")
    wid = s * NC + c

    def zrow(i, _):
        for j in range(D // 16):
            ones_v[i, pl.ds(j * 16, 16)] = jnp.zeros((16,), jnp.float32)
        return 0
    lax.fori_loop(0, K, zrow, 0)

    rowbase = s * RPT
    for b in range(RPT // K):
        pltpu.sync_copy(ones_v, acc16.at[pl.ds(rowbase + b * K, K), :])

    def orow(i, _):
        for j in range(D // 16):
            ones_v[i, pl.ds(j * 16, 16)] = jnp.ones((16,), jnp.float32)
        return 0
    lax.fori_loop(0, K, orow, 0)
    plsc.subcore_barrier()

    pltpu.sync_copy(edge_hbm.at[1, wid], dst_v)

    def chunk(j, _):
        pltpu.sync_copy(ones_v, acc16.at[dst_v.at[j]], add=True)
        return 0
    lax.fori_loop(0, NCHUNK, chunk, 0)

    plsc.subcore_barrier()
    pltpu.sync_copy(acc16.at[pl.ds(rowbase, RPT), :],
                    deg_hbm.at[c, pl.ds(rowbase, RPT), :])


_sc_deg = pl.kernel(
    _sc_deg_body,
    out_type=(jax.ShapeDtypeStruct((NC, NP, D), jnp.float32),),
    mesh=_MESH,
    scratch_types=[
        pltpu.VMEM((NCHUNK, K), jnp.int32),
        pltpu.VMEM((K, D), jnp.float32),
        pltpu.VMEM_SHARED((NP, D), jnp.float32),
    ],
)


BN = 2048
_G = NP // BN


def _tc_in_body(x_ref, w1_ref, b1_ref, w2_ref, b2_ref, feat_ref):
    h = jnp.dot(x_ref[...], w1_ref[...],
                preferred_element_type=jnp.float32) + b1_ref[...]
    h = h * jax.nn.sigmoid(h)
    feat_ref[...] = jnp.dot(h, w2_ref[...],
                            preferred_element_type=jnp.float32) + b2_ref[...]


def _tc_in(x, w1, b1, w2, b2):
    return pl.pallas_call(
        _tc_in_body,
        grid=(_G,),
        in_specs=[
            pl.BlockSpec((BN, D), lambda i: (i, 0)),
            pl.BlockSpec((D, D), lambda i: (0, 0)),
            pl.BlockSpec((1, D), lambda i: (0, 0)),
            pl.BlockSpec((D, D), lambda i: (0, 0)),
            pl.BlockSpec((1, D), lambda i: (0, 0)),
        ],
        out_specs=pl.BlockSpec((BN, D), lambda i: (i, 0)),
        out_shape=jax.ShapeDtypeStruct((NP, D), jnp.float32),
    )(x, w1, b1, w2, b2)


def _tc_comb_body(sum_deg, first, p_ref, deg_ref, w_ref, *rest):
    rest = list(rest)
    accin_ref = None if first else rest.pop(0)
    msg_ref, acc_ref = rest[0], rest[1]
    degout_ref = rest[2] if sum_deg else None

    if sum_deg:
        deg = deg_ref[0, :, :16] + deg_ref[1, :, :16]
        degout_ref[...] = deg
    else:
        deg = deg_ref[...]
    inv = 1.0 / jnp.maximum(deg[:, :1], 1.0)
    msg = (p_ref[0] + p_ref[1]) * inv
    msg_ref[...] = msg
    t = jnp.dot(msg, w_ref[...], preferred_element_type=jnp.float32)
    t = t * jax.nn.sigmoid(t)
    acc_ref[...] = t if first else accin_ref[...] + t


def _tc_combine(p, deg_or_degp, w, acc, sum_deg):
    first = acc is None
    in_specs = [
        pl.BlockSpec((NC, BN, D), lambda i: (0, i, 0)),
        (pl.BlockSpec((NC, BN, D), lambda i: (0, i, 0)) if sum_deg
         else pl.BlockSpec((BN, 16), lambda i: (i, 0))),
        pl.BlockSpec((D, D), lambda i: (0, 0)),
    ]
    args = [p, deg_or_degp, w]
    if not first:
        in_specs.append(pl.BlockSpec((BN, D), lambda i: (i, 0)))
        args.append(acc)
    out_specs = [
        pl.BlockSpec((BN, D), lambda i: (i, 0)),
        pl.BlockSpec((BN, D), lambda i: (i, 0)),
    ]
    out_shape = [
        jax.ShapeDtypeStruct((NP, D), jnp.float32),
        jax.ShapeDtypeStruct((NP, D), jnp.float32),
    ]
    if sum_deg:
        out_specs.append(pl.BlockSpec((BN, 16), lambda i: (i, 0)))
        out_shape.append(jax.ShapeDtypeStruct((NP, 16), jnp.float32))
    return pl.pallas_call(
        functools.partial(_tc_comb_body, sum_deg, first),
        grid=(_G,),
        in_specs=in_specs,
        out_specs=out_specs,
        out_shape=out_shape,
    )(*args)


def _tc_combout_body(last, p_ref, deg_ref, w_ref, acc_ref, feat_ref,
                     wo_ref, bo_ref, out_ref):
    inv = 1.0 / jnp.maximum(deg_ref[:, :1], 1.0)
    msg = (p_ref[0] + p_ref[1]) * inv
    t = jnp.dot(msg, w_ref[...], preferred_element_type=jnp.float32)
    t = t * jax.nn.sigmoid(t)
    h = jnp.dot(feat_ref[...] + acc_ref[...] + t, wo_ref[...],
                preferred_element_type=jnp.float32) + bo_ref[...]
    out_ref[...] = h if last else h * jax.nn.sigmoid(h)


def _tc_combine_out(p, deg, w, acc, feat, wo, bo, last):
    return pl.pallas_call(
        functools.partial(_tc_combout_body, last),
        grid=(_G,),
        in_specs=[
            pl.BlockSpec((NC, BN, D), lambda i: (0, i, 0)),
            pl.BlockSpec((BN, 16), lambda i: (i, 0)),
            pl.BlockSpec((D, D), lambda i: (0, 0)),
            pl.BlockSpec((BN, D), lambda i: (i, 0)),
            pl.BlockSpec((BN, D), lambda i: (i, 0)),
            pl.BlockSpec((D, D), lambda i: (0, 0)),
            pl.BlockSpec((1, D), lambda i: (0, 0)),
        ],
        out_specs=pl.BlockSpec((BN, D), lambda i: (i, 0)),
        out_shape=jax.ShapeDtypeStruct((NP, D), jnp.float32),
    )(p, deg, w, acc, feat, wo, bo)


def _tc_out_body(last, feat_ref, acc_ref, w_ref, b_ref, out_ref):
    h = jnp.dot(feat_ref[...] + acc_ref[...], w_ref[...],
                preferred_element_type=jnp.float32) + b_ref[...]
    out_ref[...] = h if last else h * jax.nn.sigmoid(h)


def _tc_out(feat, acc, w, b, last):
    return pl.pallas_call(
        functools.partial(_tc_out_body, last),
        grid=(_G,),
        in_specs=[
            pl.BlockSpec((BN, D), lambda i: (i, 0)),
            pl.BlockSpec((BN, D), lambda i: (i, 0)),
            pl.BlockSpec((D, D), lambda i: (0, 0)),
            pl.BlockSpec((1, D), lambda i: (0, 0)),
        ],
        out_specs=pl.BlockSpec((BN, D), lambda i: (i, 0)),
        out_shape=jax.ShapeDtypeStruct((NP, D), jnp.float32),
    )(feat, acc, w, b)


def kernel(x, edge_index, W_in1, b_in1, W_in2, b_in2, W_up,
           W_out0, b_out0, W_out1, b_out1):
    edge4 = edge_index.reshape(2, NW, NCHUNK, K)
    xp = jnp.pad(x, ((0, NP - N), (0, 0)))
    b1 = b_in1.reshape(1, D)
    b2 = b_in2.reshape(1, D)
    bo0 = b_out0.reshape(1, D)
    bo1 = b_out1.reshape(1, D)

    feat = _tc_in(xp, W_in1, b1, W_in2, b2)
    (degp,) = _sc_deg(edge4)

    deg = None
    for layer in range(2):
        w_out, b_out = (W_out0, bo0) if layer == 0 else (W_out1, bo1)
        msg, acc = feat, None
        for level in range(4):
            (p,) = _sc_prop(msg, edge4)
            if level == 3:
                feat = _tc_combine_out(p, deg, W_up[layer, level], acc,
                                       feat, w_out, b_out,
                                       last=(layer == 1))
            elif deg is None:
                msg, acc, deg = _tc_combine(p, degp, W_up[layer, level],
                                            acc, sum_deg=True)
            else:
                msg, acc = _tc_combine(p, deg, W_up[layer, level],
                                       acc, sum_deg=False)
    return feat[:N]
